# baseline (speedup 1.0000x reference)
import functools

import jax
import jax.numpy as jnp
from jax import lax
from jax.experimental import pallas as pl
from jax.experimental.pallas import tpu as pltpu
from jax.experimental.pallas import tpu_sc as plsc

B, V, D, NF, NNEG = 4096, 100000, 64, 3, 20
NI = 1 + NNEG
H1, H2, H3 = 256, 128, 64
TEMPERATURE = 0.02
EPS = 1e-8

HALF = 50048
PV = HALF
PBLK = 128
NPB = PV // PBLK

NC, NS = 2, 16
NW = NC * NS

U_TOT = B * NF
I_TOT = B * NI
U_PER_W = U_TOT // NW
I_PER_W = I_TOT // NW
I_CHUNK = 672


def _valid_right(bT, blk_idx):
  row = blk_idx * PBLK + lax.broadcasted_iota(jnp.int32, bT.shape, 0)
  return jnp.where(row + HALF >= V, 0.0, bT)


def _pack_user_body(a_ref, b_ref, o_ref):
  bT = _valid_right(jnp.transpose(b_ref[0]), pl.program_id(1))
  o_ref[0] = jnp.concatenate([jnp.transpose(a_ref[0]), bT], axis=1)


def _pack_user(utT):
  return pl.pallas_call(
      _pack_user_body,
      grid=(NF, NPB),
      in_specs=[
          pl.BlockSpec((1, D, PBLK), lambda f, b: (f, 0, b)),
          pl.BlockSpec((1, D, PBLK), lambda f, b: (f, 0, b + NPB)),
      ],
      out_specs=pl.BlockSpec((1, PBLK, 2 * D), lambda f, b: (f, b, 0)),
      out_shape=jax.ShapeDtypeStruct((NF, PV, 2 * D), jnp.float32),
  )(utT, utT)


def _pack_item_body(a_ref, b_ref, o_ref):
  bT = _valid_right(jnp.transpose(b_ref[...]), pl.program_id(0))
  o_ref[...] = jnp.concatenate([jnp.transpose(a_ref[...]), bT], axis=1)


def _pack_item(itT):
  return pl.pallas_call(
      _pack_item_body,
      grid=(NPB,),
      in_specs=[
          pl.BlockSpec((D, PBLK), lambda b: (0, b)),
          pl.BlockSpec((D, PBLK), lambda b: (0, b + NPB)),
      ],
      out_specs=pl.BlockSpec((PBLK, 2 * D), lambda b: (b, 0)),
      out_shape=jax.ShapeDtypeStruct((PV, 2 * D), jnp.float32),
  )(itT, itT)


def _sc_gathers(u128, uidx, i128, iidx):
  mesh = plsc.VectorSubcoreMesh(core_axis_name="c", subcore_axis_name="s")

  @functools.partial(
      pl.kernel,
      mesh=mesh,
      out_type=(jax.ShapeDtypeStruct((U_TOT, 2 * D), jnp.float32),
                jax.ShapeDtypeStruct((I_TOT, 2 * D), jnp.float32)),
      compiler_params=pltpu.CompilerParams(use_tc_tiling_on_sc=True),
      scratch_types=[
          pltpu.VMEM((I_CHUNK,), jnp.int32),
          pltpu.VMEM((I_CHUNK, 2 * D), jnp.float32),
          pltpu.SemaphoreType.DMA,
      ],
  )
  def k(ut_hbm, ui_hbm, it_hbm, ii_hbm, uo_hbm, io_hbm, idx_v, rows_v, sem):
    wid = lax.axis_index("s") * NC + lax.axis_index("c")

    ubase = wid * U_PER_W
    pltpu.sync_copy(ui_hbm.at[pl.ds(ubase, U_PER_W)],
                    idx_v.at[pl.ds(0, U_PER_W)])
    pltpu.async_copy(ut_hbm.at[idx_v.at[pl.ds(0, U_PER_W)]],
                     rows_v.at[pl.ds(0, U_PER_W)], sem).wait()
    pltpu.sync_copy(rows_v.at[pl.ds(0, U_PER_W)],
                    uo_hbm.at[pl.ds(ubase, U_PER_W)])

    ibase = wid * I_PER_W

    @pl.loop(0, I_PER_W // I_CHUNK)
    def _(ci):
      off = ibase + ci * I_CHUNK
      pltpu.sync_copy(ii_hbm.at[pl.ds(off, I_CHUNK)], idx_v)
      pltpu.async_copy(it_hbm.at[idx_v], rows_v, sem).wait()
      pltpu.sync_copy(rows_v, io_hbm.at[pl.ds(off, I_CHUNK)])

  return k(u128, uidx, i128, iidx)


BLK = 1024


def _sel_half(x128, h_col):
  lane = lax.broadcasted_iota(jnp.int32, x128.shape, 1)
  m = jnp.where(lane < D, 1.0 - h_col, h_col)
  xm = x128 * m
  return xm[:, :D] + xm[:, D:]


def _tower_body(u_ref, uh_ref, w1_ref, b1_ref, w2_ref, b2_ref, w3_ref, b3_ref,
                it_ref, ih_ref, o_ref):
  uh = uh_ref[...]
  h = b1_ref[...]
  for f in range(NF):
    uf = _sel_half(u_ref[f], uh[:, f:f + 1])
    h = h + jnp.dot(uf, w1_ref[f], preferred_element_type=jnp.float32)
  h = jnp.maximum(h, 0.0)
  h = jnp.dot(h, w2_ref[...], preferred_element_type=jnp.float32)
  h = jnp.maximum(h + b2_ref[...], 0.0)
  h = jnp.dot(h, w3_ref[...], preferred_element_type=jnp.float32)
  u = jnp.maximum(h + b3_ref[...], 0.0)
  un = jnp.sqrt(jnp.sum(u * u, axis=-1, keepdims=True))
  ih = ih_ref[...]
  cols = []
  for k in range(NI):
    itk = _sel_half(it_ref[k], ih[:, k:k + 1])
    dot = jnp.sum(u * itk, axis=-1, keepdims=True)
    inorm = jnp.sqrt(jnp.sum(itk * itk, axis=-1, keepdims=True))
    cols.append(dot / jnp.maximum(un * inorm, EPS))
  o_ref[...] = jnp.concatenate(cols, axis=1) * (1.0 / TEMPERATURE)


def _tower(u, uhalf, W1f, b1, W2, b2, W3, b3, item_rows, ihalf):
  full = lambda shape: pl.BlockSpec(shape, lambda i: (0,) * len(shape))
  return pl.pallas_call(
      _tower_body,
      grid=(B // BLK,),
      in_specs=[
          pl.BlockSpec((NF, BLK, 2 * D), lambda i: (0, i, 0)),
          pl.BlockSpec((BLK, NF), lambda i: (i, 0)),
          full((NF, D, H1)), full((1, H1)),
          full((H1, H2)), full((1, H2)),
          full((H2, H3)), full((1, H3)),
          pl.BlockSpec((NI, BLK, 2 * D), lambda i: (0, i, 0)),
          pl.BlockSpec((BLK, NI), lambda i: (i, 0)),
      ],
      out_specs=pl.BlockSpec((BLK, NI), lambda i: (i, 0)),
      out_shape=jax.ShapeDtypeStruct((B, NI), jnp.float32),
  )(u, uhalf, W1f, b1.reshape(1, H1), W2, b2.reshape(1, H2), W3,
    b3.reshape(1, H3), item_rows, ihalf)


def kernel(user_idx, pos_item_idx, neg_item_idx, user_tables, item_table,
           W1, b1, W2, b2, W3, b3):
  utT = jnp.transpose(user_tables, (0, 2, 1))
  itT = jnp.transpose(item_table, (1, 0))

  u128 = _pack_user(utT)
  i128 = _pack_item(itT)

  ui = user_idx.astype(jnp.int32).T
  uhalf_i = (ui >= HALF).astype(jnp.int32)
  uidx = ((ui - uhalf_i * HALF)
          + (jnp.arange(NF, dtype=jnp.int32) * PV)[:, None]).reshape(-1)
  uhalf = uhalf_i.T.astype(jnp.float32)

  ii_bk = jnp.concatenate(
      [pos_item_idx.astype(jnp.int32)[:, None],
       neg_item_idx.astype(jnp.int32)], axis=1)
  ihalf_bk = (ii_bk >= HALF).astype(jnp.int32)
  iidx = (ii_bk - ihalf_bk * HALF).T.reshape(-1)
  ihalf = ihalf_bk.astype(jnp.float32)

  u_rows, it_rows = _sc_gathers(u128.reshape(NF * PV, 2 * D), uidx,
                                i128, iidx)
  return _tower(u_rows.reshape(NF, B, 2 * D), uhalf,
                W1.reshape(NF, D, H1), b1, W2, b2, W3, b3,
                it_rows.reshape(NI, B, 2 * D), ihalf)


# speedup vs baseline: 2.3775x; 2.3775x over previous
import functools

import jax
import jax.numpy as jnp
from jax import lax
from jax.experimental import pallas as pl
from jax.experimental.pallas import tpu as pltpu
from jax.experimental.pallas import tpu_sc as plsc

B, V, D, NF, NNEG = 4096, 100000, 64, 3, 20
NI = 1 + NNEG
H1, H2, H3 = 256, 128, 64
TEMPERATURE = 0.02
EPS = 1e-8

HALF = 50176
PV = HALF
PBLK = 512
NPB = PV // PBLK

NC, NS = 2, 16
NW = NC * NS

U_TOT = B * NF
I_TOT = B * NI
U_PER_W = U_TOT // NW
I_PER_W = I_TOT // NW
I_CHUNK = 672


def _valid_right(bT, blk_idx):
  row = blk_idx * PBLK + lax.broadcasted_iota(jnp.int32, bT.shape, 0)
  return jnp.where(row + HALF >= V, 0.0, bT)


def _mxu_t(x):
  r = lax.broadcasted_iota(jnp.int32, (D, D), 0)
  c = lax.broadcasted_iota(jnp.int32, (D, D), 1)
  eye = (r == c).astype(jnp.float32)
  return lax.dot_general(x, eye, (((0,), (0,)), ((), ())),
                         preferred_element_type=jnp.float32)


def _pack_user_body(a_ref, b_ref, o_ref):
  bT = _valid_right(_mxu_t(b_ref[0]), pl.program_id(1))
  o_ref[0] = jnp.concatenate([_mxu_t(a_ref[0]), bT], axis=1)


def _pack_user(utT):
  return pl.pallas_call(
      _pack_user_body,
      grid=(NF, NPB),
      in_specs=[
          pl.BlockSpec((1, D, PBLK), lambda f, b: (f, 0, b)),
          pl.BlockSpec((1, D, PBLK), lambda f, b: (f, 0, b + NPB)),
      ],
      out_specs=pl.BlockSpec((1, PBLK, 2 * D), lambda f, b: (f, b, 0)),
      out_shape=jax.ShapeDtypeStruct((NF, PV, 2 * D), jnp.float32),
  )(utT, utT)


def _pack_item_body(a_ref, b_ref, o_ref):
  bT = _valid_right(_mxu_t(b_ref[...]), pl.program_id(0))
  o_ref[...] = jnp.concatenate([_mxu_t(a_ref[...]), bT], axis=1)


def _pack_item(itT):
  return pl.pallas_call(
      _pack_item_body,
      grid=(NPB,),
      in_specs=[
          pl.BlockSpec((D, PBLK), lambda b: (0, b)),
          pl.BlockSpec((D, PBLK), lambda b: (0, b + NPB)),
      ],
      out_specs=pl.BlockSpec((PBLK, 2 * D), lambda b: (b, 0)),
      out_shape=jax.ShapeDtypeStruct((PV, 2 * D), jnp.float32),
  )(itT, itT)


def _sc_gathers(u128, uidx, i128, iidx):
  mesh = plsc.VectorSubcoreMesh(core_axis_name="c", subcore_axis_name="s")

  @functools.partial(
      pl.kernel,
      mesh=mesh,
      out_type=(jax.ShapeDtypeStruct((U_TOT, 2 * D), jnp.float32),
                jax.ShapeDtypeStruct((I_TOT, 2 * D), jnp.float32)),
      compiler_params=pltpu.CompilerParams(use_tc_tiling_on_sc=True),
      scratch_types=[
          pltpu.VMEM((I_CHUNK,), jnp.int32),
          pltpu.VMEM((I_CHUNK, 2 * D), jnp.float32),
          pltpu.SemaphoreType.DMA,
      ],
  )
  def k(ut_hbm, ui_hbm, it_hbm, ii_hbm, uo_hbm, io_hbm, idx_v, rows_v, sem):
    wid = lax.axis_index("s") * NC + lax.axis_index("c")

    ubase = wid * U_PER_W
    pltpu.sync_copy(ui_hbm.at[pl.ds(ubase, U_PER_W)],
                    idx_v.at[pl.ds(0, U_PER_W)])
    pltpu.async_copy(ut_hbm.at[idx_v.at[pl.ds(0, U_PER_W)]],
                     rows_v.at[pl.ds(0, U_PER_W)], sem).wait()
    pltpu.sync_copy(rows_v.at[pl.ds(0, U_PER_W)],
                    uo_hbm.at[pl.ds(ubase, U_PER_W)])

    ibase = wid * I_PER_W

    @pl.loop(0, I_PER_W // I_CHUNK)
    def _(ci):
      off = ibase + ci * I_CHUNK
      pltpu.sync_copy(ii_hbm.at[pl.ds(off, I_CHUNK)], idx_v)
      pltpu.async_copy(it_hbm.at[idx_v], rows_v, sem).wait()
      pltpu.sync_copy(rows_v, io_hbm.at[pl.ds(off, I_CHUNK)])

  return k(u128, uidx, i128, iidx)


BLK = 1024


def _sel_half(x128, h_col):
  lane = lax.broadcasted_iota(jnp.int32, x128.shape, 1)
  m = jnp.where(lane < D, 1.0 - h_col, h_col)
  xm = x128 * m
  return xm[:, :D] + xm[:, D:]


def _tower_body(u_ref, uh_ref, w1_ref, b1_ref, w2_ref, b2_ref, w3_ref, b3_ref,
                it_ref, ih_ref, o_ref):
  uh = uh_ref[...]
  h = b1_ref[...]
  for f in range(NF):
    uf = _sel_half(u_ref[f], uh[:, f:f + 1])
    h = h + jnp.dot(uf, w1_ref[f], preferred_element_type=jnp.float32)
  h = jnp.maximum(h, 0.0)
  h = jnp.dot(h, w2_ref[...], preferred_element_type=jnp.float32)
  h = jnp.maximum(h + b2_ref[...], 0.0)
  h = jnp.dot(h, w3_ref[...], preferred_element_type=jnp.float32)
  u = jnp.maximum(h + b3_ref[...], 0.0)
  un = jnp.sqrt(jnp.sum(u * u, axis=-1, keepdims=True))
  ih = ih_ref[...]
  cols = []
  for k in range(NI):
    itk = _sel_half(it_ref[k], ih[:, k:k + 1])
    dot = jnp.sum(u * itk, axis=-1, keepdims=True)
    inorm = jnp.sqrt(jnp.sum(itk * itk, axis=-1, keepdims=True))
    cols.append(dot / jnp.maximum(un * inorm, EPS))
  o_ref[...] = jnp.concatenate(cols, axis=1) * (1.0 / TEMPERATURE)


def _tower(u, uhalf, W1f, b1, W2, b2, W3, b3, item_rows, ihalf):
  full = lambda shape: pl.BlockSpec(shape, lambda i: (0,) * len(shape))
  return pl.pallas_call(
      _tower_body,
      grid=(B // BLK,),
      in_specs=[
          pl.BlockSpec((NF, BLK, 2 * D), lambda i: (0, i, 0)),
          pl.BlockSpec((BLK, NF), lambda i: (i, 0)),
          full((NF, D, H1)), full((1, H1)),
          full((H1, H2)), full((1, H2)),
          full((H2, H3)), full((1, H3)),
          pl.BlockSpec((NI, BLK, 2 * D), lambda i: (0, i, 0)),
          pl.BlockSpec((BLK, NI), lambda i: (i, 0)),
      ],
      out_specs=pl.BlockSpec((BLK, NI), lambda i: (i, 0)),
      out_shape=jax.ShapeDtypeStruct((B, NI), jnp.float32),
  )(u, uhalf, W1f, b1.reshape(1, H1), W2, b2.reshape(1, H2), W3,
    b3.reshape(1, H3), item_rows, ihalf)


def kernel(user_idx, pos_item_idx, neg_item_idx, user_tables, item_table,
           W1, b1, W2, b2, W3, b3):
  utT = jnp.transpose(user_tables, (0, 2, 1))
  itT = jnp.transpose(item_table, (1, 0))

  u128 = _pack_user(utT)
  i128 = _pack_item(itT)

  ui = user_idx.astype(jnp.int32).T
  uhalf_i = (ui >= HALF).astype(jnp.int32)
  uidx = ((ui - uhalf_i * HALF)
          + (jnp.arange(NF, dtype=jnp.int32) * PV)[:, None]).reshape(-1)
  uhalf = uhalf_i.T.astype(jnp.float32)

  ii_bk = jnp.concatenate(
      [pos_item_idx.astype(jnp.int32)[:, None],
       neg_item_idx.astype(jnp.int32)], axis=1)
  ihalf_bk = (ii_bk >= HALF).astype(jnp.int32)
  iidx = (ii_bk - ihalf_bk * HALF).T.reshape(-1)
  ihalf = ihalf_bk.astype(jnp.float32)

  u_rows, it_rows = _sc_gathers(u128.reshape(NF * PV, 2 * D), uidx,
                                i128, iidx)
  return _tower(u_rows.reshape(NF, B, 2 * D), uhalf,
                W1.reshape(NF, D, H1), b1, W2, b2, W3, b3,
                it_rows.reshape(NI, B, 2 * D), ihalf)


# speedup vs baseline: 3.7287x; 1.5684x over previous
import functools

import jax
import jax.numpy as jnp
from jax import lax
from jax.experimental import pallas as pl
from jax.experimental.pallas import tpu as pltpu
from jax.experimental.pallas import tpu_sc as plsc

B, V, D, NF, NNEG = 4096, 100000, 64, 3, 20
NI = 1 + NNEG
H1, H2, H3 = 256, 128, 64
TEMPERATURE = 0.02
EPS = 1e-8

HALF = 51200
PV = HALF
PBLK = 2048
NPB = PV // PBLK

NC, NS = 2, 16
NW = NC * NS

U_TOT = B * NF
I_TOT = B * NI
U_PER_W = U_TOT // NW
I_PER_W = I_TOT // NW
I_CHUNK = 672


def _valid_right(bT, blk_idx):
  row = blk_idx * PBLK + lax.broadcasted_iota(jnp.int32, bT.shape, 0)
  return jnp.where(row + HALF >= V, 0.0, bT)


def _mxu_t(x):
  r = lax.broadcasted_iota(jnp.int32, (D, D), 0)
  c = lax.broadcasted_iota(jnp.int32, (D, D), 1)
  eye = (r == c).astype(jnp.float32)
  return lax.dot_general(x, eye, (((0,), (0,)), ((), ())),
                         preferred_element_type=jnp.float32)


def _pack_user_body(a_ref, b_ref, o_ref):
  bT = _valid_right(_mxu_t(b_ref[0]), pl.program_id(1))
  o_ref[0] = jnp.concatenate([_mxu_t(a_ref[0]), bT], axis=1)


MAXB = (V - 1) // PBLK


def _pack_user(utT):
  return pl.pallas_call(
      _pack_user_body,
      grid=(NF, NPB),
      in_specs=[
          pl.BlockSpec((1, D, PBLK), lambda f, b: (f, 0, b)),
          pl.BlockSpec((1, D, PBLK),
                       lambda f, b: (f, 0, jnp.minimum(b + NPB, MAXB))),
      ],
      out_specs=pl.BlockSpec((1, PBLK, 2 * D), lambda f, b: (f, b, 0)),
      out_shape=jax.ShapeDtypeStruct((NF, PV, 2 * D), jnp.float32),
  )(utT, utT)


def _pack_item_body(a_ref, b_ref, o_ref):
  bT = _valid_right(_mxu_t(b_ref[...]), pl.program_id(0))
  o_ref[...] = jnp.concatenate([_mxu_t(a_ref[...]), bT], axis=1)


def _pack_item(itT):
  return pl.pallas_call(
      _pack_item_body,
      grid=(NPB,),
      in_specs=[
          pl.BlockSpec((D, PBLK), lambda b: (0, b)),
          pl.BlockSpec((D, PBLK),
                       lambda b: (0, jnp.minimum(b + NPB, MAXB))),
      ],
      out_specs=pl.BlockSpec((PBLK, 2 * D), lambda b: (b, 0)),
      out_shape=jax.ShapeDtypeStruct((PV, 2 * D), jnp.float32),
  )(itT, itT)


def _sc_gathers(u128, uidx, i128, iidx):
  mesh = plsc.VectorSubcoreMesh(core_axis_name="c", subcore_axis_name="s")

  @functools.partial(
      pl.kernel,
      mesh=mesh,
      out_type=(jax.ShapeDtypeStruct((U_TOT, 2 * D), jnp.float32),
                jax.ShapeDtypeStruct((I_TOT, 2 * D), jnp.float32)),
      compiler_params=pltpu.CompilerParams(use_tc_tiling_on_sc=True),
      scratch_types=[
          pltpu.VMEM((I_CHUNK,), jnp.int32),
          pltpu.VMEM((I_CHUNK, 2 * D), jnp.float32),
          pltpu.SemaphoreType.DMA,
      ],
  )
  def k(ut_hbm, ui_hbm, it_hbm, ii_hbm, uo_hbm, io_hbm, idx_v, rows_v, sem):
    wid = lax.axis_index("s") * NC + lax.axis_index("c")

    ubase = wid * U_PER_W
    pltpu.sync_copy(ui_hbm.at[pl.ds(ubase, U_PER_W)],
                    idx_v.at[pl.ds(0, U_PER_W)])
    pltpu.async_copy(ut_hbm.at[idx_v.at[pl.ds(0, U_PER_W)]],
                     rows_v.at[pl.ds(0, U_PER_W)], sem).wait()
    pltpu.sync_copy(rows_v.at[pl.ds(0, U_PER_W)],
                    uo_hbm.at[pl.ds(ubase, U_PER_W)])

    ibase = wid * I_PER_W

    @pl.loop(0, I_PER_W // I_CHUNK)
    def _(ci):
      off = ibase + ci * I_CHUNK
      pltpu.sync_copy(ii_hbm.at[pl.ds(off, I_CHUNK)], idx_v)
      pltpu.async_copy(it_hbm.at[idx_v], rows_v, sem).wait()
      pltpu.sync_copy(rows_v, io_hbm.at[pl.ds(off, I_CHUNK)])

  return k(u128, uidx, i128, iidx)


BLK = 1024


def _sel_half(x128, h_col):
  lane = lax.broadcasted_iota(jnp.int32, x128.shape, 1)
  m = jnp.where(lane < D, 1.0 - h_col, h_col)
  xm = x128 * m
  return xm[:, :D] + xm[:, D:]


def _tower_body(u_ref, uh_ref, w1_ref, b1_ref, w2_ref, b2_ref, w3_ref, b3_ref,
                it_ref, ih_ref, o_ref):
  uh = uh_ref[...]
  h = b1_ref[...]
  for f in range(NF):
    uf = _sel_half(u_ref[f], uh[:, f:f + 1])
    h = h + jnp.dot(uf, w1_ref[f], preferred_element_type=jnp.float32)
  h = jnp.maximum(h, 0.0)
  h = jnp.dot(h, w2_ref[...], preferred_element_type=jnp.float32)
  h = jnp.maximum(h + b2_ref[...], 0.0)
  h = jnp.dot(h, w3_ref[...], preferred_element_type=jnp.float32)
  u = jnp.maximum(h + b3_ref[...], 0.0)
  un = jnp.sqrt(jnp.sum(u * u, axis=-1, keepdims=True))
  ih = ih_ref[...]
  cols = []
  for k in range(NI):
    itk = _sel_half(it_ref[k], ih[:, k:k + 1])
    dot = jnp.sum(u * itk, axis=-1, keepdims=True)
    inorm = jnp.sqrt(jnp.sum(itk * itk, axis=-1, keepdims=True))
    cols.append(dot / jnp.maximum(un * inorm, EPS))
  o_ref[...] = jnp.concatenate(cols, axis=1) * (1.0 / TEMPERATURE)


def _tower(u, uhalf, W1f, b1, W2, b2, W3, b3, item_rows, ihalf):
  full = lambda shape: pl.BlockSpec(shape, lambda i: (0,) * len(shape))
  return pl.pallas_call(
      _tower_body,
      grid=(B // BLK,),
      in_specs=[
          pl.BlockSpec((NF, BLK, 2 * D), lambda i: (0, i, 0)),
          pl.BlockSpec((BLK, NF), lambda i: (i, 0)),
          full((NF, D, H1)), full((1, H1)),
          full((H1, H2)), full((1, H2)),
          full((H2, H3)), full((1, H3)),
          pl.BlockSpec((NI, BLK, 2 * D), lambda i: (0, i, 0)),
          pl.BlockSpec((BLK, NI), lambda i: (i, 0)),
      ],
      out_specs=pl.BlockSpec((BLK, NI), lambda i: (i, 0)),
      out_shape=jax.ShapeDtypeStruct((B, NI), jnp.float32),
  )(u, uhalf, W1f, b1.reshape(1, H1), W2, b2.reshape(1, H2), W3,
    b3.reshape(1, H3), item_rows, ihalf)


def kernel(user_idx, pos_item_idx, neg_item_idx, user_tables, item_table,
           W1, b1, W2, b2, W3, b3):
  utT = jnp.transpose(user_tables, (0, 2, 1))
  itT = jnp.transpose(item_table, (1, 0))

  u128 = _pack_user(utT)
  i128 = _pack_item(itT)

  ui = user_idx.astype(jnp.int32).T
  uhalf_i = (ui >= HALF).astype(jnp.int32)
  uidx = ((ui - uhalf_i * HALF)
          + (jnp.arange(NF, dtype=jnp.int32) * PV)[:, None]).reshape(-1)
  uhalf = uhalf_i.T.astype(jnp.float32)

  ii_bk = jnp.concatenate(
      [pos_item_idx.astype(jnp.int32)[:, None],
       neg_item_idx.astype(jnp.int32)], axis=1)
  ihalf_bk = (ii_bk >= HALF).astype(jnp.int32)
  iidx = (ii_bk - ihalf_bk * HALF).T.reshape(-1)
  ihalf = ihalf_bk.astype(jnp.float32)

  u_rows, it_rows = _sc_gathers(u128.reshape(NF * PV, 2 * D), uidx,
                                i128, iidx)
  return _tower(u_rows.reshape(NF, B, 2 * D), uhalf,
                W1.reshape(NF, D, H1), b1, W2, b2, W3, b3,
                it_rows.reshape(NI, B, 2 * D), ihalf)


# speedup vs baseline: 3.9618x; 1.0625x over previous
import functools

import jax
import jax.numpy as jnp
from jax import lax
from jax.experimental import pallas as pl
from jax.experimental.pallas import tpu as pltpu
from jax.experimental.pallas import tpu_sc as plsc

B, V, D, NF, NNEG = 4096, 100000, 64, 3, 20
NI = 1 + NNEG
H1, H2, H3 = 256, 128, 64
TEMPERATURE = 0.02
EPS = 1e-8

HALF = 51200
PV = HALF
PBLK = 2048
NPB = PV // PBLK
MAXB = (V - 1) // PBLK

NC, NS = 2, 16
NW = NC * NS

U_TOT = B * NF
I_TOT = B * NI
U_PER_W = U_TOT // NW
I_PER_W = I_TOT // NW
I_CHUNK = 672


def _valid_right(bT, blk_idx):
  row = blk_idx * PBLK + lax.broadcasted_iota(jnp.int32, bT.shape, 0)
  return jnp.where(row + HALF >= V, 0.0, bT)


def _mxu_t(x):
  r = lax.broadcasted_iota(jnp.int32, (D, D), 0)
  c = lax.broadcasted_iota(jnp.int32, (D, D), 1)
  eye = (r == c).astype(jnp.float32)
  return lax.dot_general(x, eye, (((0,), (0,)), ((), ())),
                         preferred_element_type=jnp.float32)


def _pack_user_body(a_ref, b_ref, o_ref):
  bT = _valid_right(_mxu_t(b_ref[0]), pl.program_id(1))
  o_ref[0] = jnp.concatenate([_mxu_t(a_ref[0]), bT], axis=1)


def _pack_user(utT):
  return pl.pallas_call(
      _pack_user_body,
      grid=(NF, NPB),
      in_specs=[
          pl.BlockSpec((1, D, PBLK), lambda f, b: (f, 0, b)),
          pl.BlockSpec((1, D, PBLK),
                       lambda f, b: (f, 0, jnp.minimum(b + NPB, MAXB))),
      ],
      out_specs=pl.BlockSpec((1, PBLK, 2 * D), lambda f, b: (f, b, 0)),
      out_shape=jax.ShapeDtypeStruct((NF, PV, 2 * D), jnp.float32),
  )(utT, utT)


def _pack_item_body(a_ref, b_ref, o_ref):
  bT = _valid_right(_mxu_t(b_ref[...]), pl.program_id(0))
  o_ref[...] = jnp.concatenate([_mxu_t(a_ref[...]), bT], axis=1)


def _pack_item(itT):
  return pl.pallas_call(
      _pack_item_body,
      grid=(NPB,),
      in_specs=[
          pl.BlockSpec((D, PBLK), lambda b: (0, b)),
          pl.BlockSpec((D, PBLK),
                       lambda b: (0, jnp.minimum(b + NPB, MAXB))),
      ],
      out_specs=pl.BlockSpec((PBLK, 2 * D), lambda b: (b, 0)),
      out_shape=jax.ShapeDtypeStruct((PV, 2 * D), jnp.float32),
  )(itT, itT)


def _sc_gather(table, idx, total, per_w, chunk):
  mesh = plsc.VectorSubcoreMesh(core_axis_name="c", subcore_axis_name="s")
  n_chunks = per_w // chunk

  @functools.partial(
      pl.kernel,
      mesh=mesh,
      out_type=jax.ShapeDtypeStruct((total, 2 * D), jnp.float32),
      compiler_params=pltpu.CompilerParams(use_tc_tiling_on_sc=True),
      scratch_types=[
          pltpu.VMEM((chunk,), jnp.int32),
          pltpu.VMEM((chunk, 2 * D), jnp.float32),
          pltpu.SemaphoreType.DMA,
      ],
  )
  def k(t_hbm, i_hbm, o_hbm, idx_v, rows_v, sem):
    wid = lax.axis_index("s") * NC + lax.axis_index("c")
    base = wid * per_w

    @pl.loop(0, n_chunks)
    def _(ci):
      off = base + ci * chunk
      pltpu.sync_copy(i_hbm.at[pl.ds(off, chunk)], idx_v)
      pltpu.async_copy(t_hbm.at[idx_v], rows_v, sem).wait()
      pltpu.sync_copy(rows_v, o_hbm.at[pl.ds(off, chunk)])

  return k(table, idx)


BLK = 1024


def _sel_half(x128, h_col):
  lane = lax.broadcasted_iota(jnp.int32, x128.shape, 1)
  m = jnp.where(lane < D, 1.0 - h_col, h_col)
  xm = x128 * m
  return xm[:, :D] + xm[:, D:]


def _tower_body(u_ref, uh_ref, w1_ref, b1_ref, w2_ref, b2_ref, w3_ref, b3_ref,
                it_ref, ih_ref, o_ref):
  uh = uh_ref[...]
  h = b1_ref[...]
  for f in range(NF):
    uf = _sel_half(u_ref[f], uh[:, f:f + 1])
    h = h + jnp.dot(uf, w1_ref[f], preferred_element_type=jnp.float32)
  h = jnp.maximum(h, 0.0)
  h = jnp.dot(h, w2_ref[...], preferred_element_type=jnp.float32)
  h = jnp.maximum(h + b2_ref[...], 0.0)
  h = jnp.dot(h, w3_ref[...], preferred_element_type=jnp.float32)
  u = jnp.maximum(h + b3_ref[...], 0.0)
  usq = jnp.sum(u * u, axis=-1, keepdims=True)
  udup = jnp.concatenate([u, u], axis=1)
  lane = lax.broadcasted_iota(jnp.int32, (BLK, 2 * D), 1)
  ih = ih_ref[...]
  cols = []
  for k in range(NI):
    h_col = ih[:, k:k + 1]
    m = jnp.where(lane < D, 1.0 - h_col, h_col)
    p = it_ref[k] * m
    dot = jnp.sum(udup * p, axis=-1, keepdims=True)
    isq = jnp.sum(p * p, axis=-1, keepdims=True)
    cols.append(dot / jnp.maximum(jnp.sqrt(usq * isq), EPS))
  o_ref[...] = jnp.concatenate(cols, axis=1) * (1.0 / TEMPERATURE)


def _tower(u, uhalf, W1f, b1, W2, b2, W3, b3, item_rows, ihalf):
  full = lambda shape: pl.BlockSpec(shape, lambda i: (0,) * len(shape))
  return pl.pallas_call(
      _tower_body,
      grid=(B // BLK,),
      in_specs=[
          pl.BlockSpec((NF, BLK, 2 * D), lambda i: (0, i, 0)),
          pl.BlockSpec((BLK, NF), lambda i: (i, 0)),
          full((NF, D, H1)), full((1, H1)),
          full((H1, H2)), full((1, H2)),
          full((H2, H3)), full((1, H3)),
          pl.BlockSpec((NI, BLK, 2 * D), lambda i: (0, i, 0)),
          pl.BlockSpec((BLK, NI), lambda i: (i, 0)),
      ],
      out_specs=pl.BlockSpec((BLK, NI), lambda i: (i, 0)),
      out_shape=jax.ShapeDtypeStruct((B, NI), jnp.float32),
  )(u, uhalf, W1f, b1.reshape(1, H1), W2, b2.reshape(1, H2), W3,
    b3.reshape(1, H3), item_rows, ihalf)


def kernel(user_idx, pos_item_idx, neg_item_idx, user_tables, item_table,
           W1, b1, W2, b2, W3, b3):
  itT = jnp.transpose(item_table, (1, 0))
  utT = jnp.transpose(user_tables, (0, 2, 1))

  ii_bk = jnp.concatenate(
      [pos_item_idx.astype(jnp.int32)[:, None],
       neg_item_idx.astype(jnp.int32)], axis=1)
  ihalf_bk = (ii_bk >= HALF).astype(jnp.int32)
  iidx = (ii_bk - ihalf_bk * HALF).T.reshape(-1)
  ihalf = ihalf_bk.astype(jnp.float32)

  ui = user_idx.astype(jnp.int32).T
  uhalf_i = (ui >= HALF).astype(jnp.int32)
  uidx = ((ui - uhalf_i * HALF)
          + (jnp.arange(NF, dtype=jnp.int32) * PV)[:, None]).reshape(-1)
  uhalf = uhalf_i.T.astype(jnp.float32)

  i128 = _pack_item(itT)
  it_rows = _sc_gather(i128, iidx, I_TOT, I_PER_W, I_CHUNK)
  u128 = _pack_user(utT)
  u_rows = _sc_gather(u128.reshape(NF * PV, 2 * D), uidx,
                      U_TOT, U_PER_W, U_PER_W)

  return _tower(u_rows.reshape(NF, B, 2 * D), uhalf,
                W1.reshape(NF, D, H1), b1, W2, b2, W3, b3,
                it_rows.reshape(NI, B, 2 * D), ihalf)


# speedup vs baseline: 4.2291x; 1.0675x over previous
import functools

import jax
import jax.numpy as jnp
from jax import lax
from jax.experimental import pallas as pl
from jax.experimental.pallas import tpu as pltpu
from jax.experimental.pallas import tpu_sc as plsc

B, V, D, NF, NNEG = 4096, 100000, 64, 3, 20
NI = 1 + NNEG
H1, H2, H3 = 256, 128, 64
TEMPERATURE = 0.02
EPS = 1e-8

HALF = 51200
PV = HALF
PBLK = 2048
NPB = PV // PBLK
MAXB = (V - 1) // PBLK

NC, NS = 2, 16
NW = NC * NS

U_TOT = B * NF
I_TOT = B * NI
U_PER_W = U_TOT // NW
I_PER_W = I_TOT // NW
I_CHUNK = 672


def _pack_pair(a, b, blk_idx):
  x = jnp.concatenate([a, b], axis=0)
  r = lax.broadcasted_iota(jnp.int32, (2 * D, 2 * D), 0)
  c = lax.broadcasted_iota(jnp.int32, (2 * D, 2 * D), 1)
  eye = (r == c).astype(jnp.float32)
  xT = lax.dot_general(x, eye, (((0,), (0,)), ((), ())),
                       preferred_element_type=jnp.float32)
  row = blk_idx * PBLK + lax.broadcasted_iota(jnp.int32, xT.shape, 0)
  lane = lax.broadcasted_iota(jnp.int32, xT.shape, 1)
  return jnp.where((lane >= D) & (row + HALF >= V), 0.0, xT)


def _pack_user_body(i128_ref, a_ref, b_ref, o_ref):
  del i128_ref
  o_ref[0] = _pack_pair(a_ref[0], b_ref[0], pl.program_id(1))


def _pack_user(utT, i128):
  return pl.pallas_call(
      _pack_user_body,
      grid=(NF, NPB),
      in_specs=[
          pl.BlockSpec((8, 2 * D), lambda f, b: (0, 0)),
          pl.BlockSpec((1, D, PBLK), lambda f, b: (f, 0, b)),
          pl.BlockSpec((1, D, PBLK),
                       lambda f, b: (f, 0, jnp.minimum(b + NPB, MAXB))),
      ],
      out_specs=pl.BlockSpec((1, PBLK, 2 * D), lambda f, b: (f, b, 0)),
      out_shape=jax.ShapeDtypeStruct((NF, PV, 2 * D), jnp.float32),
  )(i128, utT, utT)


def _pack_item_body(a_ref, b_ref, o_ref):
  o_ref[...] = _pack_pair(a_ref[...], b_ref[...], pl.program_id(0))


def _pack_item(itT):
  return pl.pallas_call(
      _pack_item_body,
      grid=(NPB,),
      in_specs=[
          pl.BlockSpec((D, PBLK), lambda b: (0, b)),
          pl.BlockSpec((D, PBLK),
                       lambda b: (0, jnp.minimum(b + NPB, MAXB))),
      ],
      out_specs=pl.BlockSpec((PBLK, 2 * D), lambda b: (b, 0)),
      out_shape=jax.ShapeDtypeStruct((PV, 2 * D), jnp.float32),
  )(itT, itT)


def _sc_gather(table, idx, total, per_w, chunk):
  mesh = plsc.VectorSubcoreMesh(core_axis_name="c", subcore_axis_name="s")
  n_chunks = per_w // chunk

  @functools.partial(
      pl.kernel,
      mesh=mesh,
      out_type=jax.ShapeDtypeStruct((total, 2 * D), jnp.float32),
      compiler_params=pltpu.CompilerParams(use_tc_tiling_on_sc=True),
      scratch_types=[
          pltpu.VMEM((chunk,), jnp.int32),
          pltpu.VMEM((chunk, 2 * D), jnp.float32),
          pltpu.SemaphoreType.DMA,
      ],
  )
  def k(t_hbm, i_hbm, o_hbm, idx_v, rows_v, sem):
    wid = lax.axis_index("s") * NC + lax.axis_index("c")
    base = wid * per_w

    @pl.loop(0, n_chunks)
    def _(ci):
      off = base + ci * chunk
      pltpu.sync_copy(i_hbm.at[pl.ds(off, chunk)], idx_v)
      pltpu.async_copy(t_hbm.at[idx_v], rows_v, sem).wait()
      pltpu.sync_copy(rows_v, o_hbm.at[pl.ds(off, chunk)])

  return k(table, idx)


BLK = 1024


def _sel_half(x128, h_col):
  lane = lax.broadcasted_iota(jnp.int32, x128.shape, 1)
  m = jnp.where(lane < D, 1.0 - h_col, h_col)
  xm = x128 * m
  return xm[:, :D] + xm[:, D:]


def _tower_body(u_ref, uh_ref, w1_ref, b1_ref, w2_ref, b2_ref, w3_ref, b3_ref,
                it_ref, ih_ref, o_ref):
  uh = uh_ref[...]
  h = b1_ref[...]
  for f in range(NF):
    uf = _sel_half(u_ref[f], uh[:, f:f + 1])
    h = h + jnp.dot(uf, w1_ref[f], preferred_element_type=jnp.float32)
  h = jnp.maximum(h, 0.0)
  h = jnp.dot(h, w2_ref[...], preferred_element_type=jnp.float32)
  h = jnp.maximum(h + b2_ref[...], 0.0)
  h = jnp.dot(h, w3_ref[...], preferred_element_type=jnp.float32)
  u = jnp.maximum(h + b3_ref[...], 0.0)
  usq = jnp.sum(u * u, axis=-1, keepdims=True)
  udup = jnp.concatenate([u, u], axis=1)
  lane = lax.broadcasted_iota(jnp.int32, (BLK, 2 * D), 1)
  ih = ih_ref[...]
  cols = []
  for k in range(NI):
    h_col = ih[:, k:k + 1]
    m = jnp.where(lane < D, 1.0 - h_col, h_col)
    p = it_ref[k] * m
    dot = jnp.sum(udup * p, axis=-1, keepdims=True)
    isq = jnp.sum(p * p, axis=-1, keepdims=True)
    cols.append(dot / jnp.maximum(jnp.sqrt(usq * isq), EPS))
  o_ref[...] = jnp.concatenate(cols, axis=1) * (1.0 / TEMPERATURE)


def _tower(u, uhalf, W1f, b1, W2, b2, W3, b3, item_rows, ihalf):
  full = lambda shape: pl.BlockSpec(shape, lambda i: (0,) * len(shape))
  return pl.pallas_call(
      _tower_body,
      grid=(B // BLK,),
      in_specs=[
          pl.BlockSpec((NF, BLK, 2 * D), lambda i: (0, i, 0)),
          pl.BlockSpec((BLK, NF), lambda i: (i, 0)),
          full((NF, D, H1)), full((1, H1)),
          full((H1, H2)), full((1, H2)),
          full((H2, H3)), full((1, H3)),
          pl.BlockSpec((NI, BLK, 2 * D), lambda i: (0, i, 0)),
          pl.BlockSpec((BLK, NI), lambda i: (i, 0)),
      ],
      out_specs=pl.BlockSpec((BLK, NI), lambda i: (i, 0)),
      out_shape=jax.ShapeDtypeStruct((B, NI), jnp.float32),
  )(u, uhalf, W1f, b1.reshape(1, H1), W2, b2.reshape(1, H2), W3,
    b3.reshape(1, H3), item_rows, ihalf)


def kernel(user_idx, pos_item_idx, neg_item_idx, user_tables, item_table,
           W1, b1, W2, b2, W3, b3):
  itT = jnp.transpose(item_table, (1, 0))
  utT = jnp.transpose(user_tables, (0, 2, 1))

  ii_bk = jnp.concatenate(
      [pos_item_idx.astype(jnp.int32)[:, None],
       neg_item_idx.astype(jnp.int32)], axis=1)
  ihalf_bk = (ii_bk >= HALF).astype(jnp.int32)
  iidx = (ii_bk - ihalf_bk * HALF).T.reshape(-1)
  ihalf = ihalf_bk.astype(jnp.float32)

  ui = user_idx.astype(jnp.int32).T
  uhalf_i = (ui >= HALF).astype(jnp.int32)
  uidx = ((ui - uhalf_i * HALF)
          + (jnp.arange(NF, dtype=jnp.int32) * PV)[:, None]).reshape(-1)
  uhalf = uhalf_i.T.astype(jnp.float32)

  i128 = _pack_item(itT)
  it_rows = _sc_gather(i128, iidx, I_TOT, I_PER_W, I_CHUNK)
  u128 = _pack_user(utT, i128)
  u_rows = _sc_gather(u128.reshape(NF * PV, 2 * D), uidx,
                      U_TOT, U_PER_W, U_PER_W)

  return _tower(u_rows.reshape(NF, B, 2 * D), uhalf,
                W1.reshape(NF, D, H1), b1, W2, b2, W3, b3,
                it_rows.reshape(NI, B, 2 * D), ihalf)
